# baseline (speedup 1.0000x reference)
import jax
import jax.numpy as jnp
from jax import lax
from jax.experimental import pallas as pl
from jax.experimental.pallas import tpu as pltpu

EMB = 64


def _relu(v):
    return jnp.maximum(v, 0.0)


def _dot(a, b):
    return lax.dot_general(a, b, (((1,), (0,)), ((), ())),
                           preferred_element_type=jnp.float32)


def _encode_nodes_body(x_ref, w1_ref, b1_ref, w2_ref, b2_ref, h2_ref):
    h = _relu(_dot(x_ref[...], w1_ref[...]) + b1_ref[...])
    h = _relu(_dot(h, w2_ref[...]) + b2_ref[...])
    h2_ref[:, :EMB] = h
    h2_ref[:, EMB:] = h


def _encode_edges_body(a_ref, w1_ref, b1_ref, w2_ref, b2_ref, c2_ref, bc_ref,
                       ea_ref, eac_ref):
    e = _relu(_dot(a_ref[...], w1_ref[...]) + b1_ref[...])
    e = _relu(_dot(e, w2_ref[...]) + b2_ref[...])
    ea_ref[...] = e
    eac_ref[...] = _dot(e, c2_ref[...]) + bc_ref[...]


def _proj_body(h2_ref, wsd_ref, wb_ref, ps_ref, pd_ref):
    h2 = h2_ref[...]
    ps_ref[...] = _dot(h2, wsd_ref[...])
    pd_ref[...] = _dot(h2, wb_ref[...])


def _edge_mlp_body(gs_ref, gd_ref, ea_ref, eac_ref,
                   c1_ref, w2_ref, b2_ref, f_ref, b3_ref, w4_ref, b4_ref,
                   eao_ref, m_ref):
    gs = gs_ref[...]
    t = _relu(gs[:, :EMB] + gd_ref[...]
              + _dot(ea_ref[...], c1_ref[...]) + eac_ref[...])
    ea_new = _relu(_dot(t, w2_ref[...]) + b2_ref[...])
    u = _relu(gs[:, EMB:] + _dot(ea_new, f_ref[...]) + b3_ref[...])
    m_ref[...] = _relu(_dot(u, w4_ref[...]) + b4_ref[...])
    eao_ref[...] = ea_new


def _node_update_body(h2_ref, agg_ref, wg_ref, wh_ref, ba_ref, w2_ref, b2_ref,
                      h2o_ref):
    z = _relu(_dot(h2_ref[...], wg_ref[...])
              + _dot(agg_ref[...], wh_ref[...]) + ba_ref[...])
    hn = _relu(_dot(z, w2_ref[...]) + b2_ref[...])
    h2o_ref[:, :EMB] = hn
    h2o_ref[:, EMB:] = h2_ref[:, EMB:]


def _decoder_body(h_ref, w3_ref, b3_ref, w4_ref, b4_ref, w5_ref, b5_ref,
                  o_ref):
    o = _relu(_dot(h_ref[:, :EMB], w3_ref[...]) + b3_ref[...])
    o = _relu(_dot(o, w4_ref[...]) + b4_ref[...])
    o_ref[...] = _dot(o, w5_ref[...]) + b5_ref[...]


def _row_call(body, n, bn, blocked, weights, out_cols):
    grid = (n // bn,)
    in_specs = [pl.BlockSpec((bn, bc), (lambda ci: (lambda i: (i, ci)))(ci))
                for (_, bc, ci) in blocked]
    in_specs += [pl.BlockSpec(w.shape, (lambda nd: (lambda i: (0,) * nd))(w.ndim))
                 for w in weights]
    out_specs = [pl.BlockSpec((bn, c), lambda i: (i, 0)) for c in out_cols]
    out_shape = [jax.ShapeDtypeStruct((n, c), jnp.float32) for c in out_cols]
    single = len(out_cols) == 1
    res = pl.pallas_call(
        body,
        grid=grid,
        in_specs=in_specs,
        out_specs=out_specs[0] if single else out_specs,
        out_shape=out_shape[0] if single else out_shape,
    )(*[a for (a, _, _) in blocked], *weights)
    return res


def kernel(x, edge_index, edge_attr, params):
    N = x.shape[0]
    E = edge_index.shape[1]
    BN = 2000 if N % 2000 == 0 else N
    BE = 2000 if E % 2000 == 0 else E
    send = edge_index[0]
    recv = edge_index[1]

    W1, b1 = params['fc1']
    W2, b2 = params['fc2']
    We1, be1 = params['efc1']
    We2, be2 = params['efc2']
    Wem1, bem1 = params['em1']
    Wem2, bem2 = params['em2']
    Wnm1a, bnm1a = params['nm1a']
    Wnm1b, bnm1b = params['nm1b']
    Wnm2a, bnm2a = params['nm2a']
    Wnm2b, bnm2b = params['nm2b']
    W3, b3 = params['fc3']
    W4, b4 = params['fc4']
    W5, b5 = params['fc5']

    A, B, C1, C2 = Wem1[0:128], Wem1[128:256], Wem1[256:320], Wem1[320:384]
    D, F = Wnm1a[0:128], Wnm1a[128:192]
    Wsd = jnp.concatenate([A, D], axis=1)
    G, H = Wnm2a[0:128], Wnm2a[128:192]

    W1p = jnp.zeros((8, EMB), jnp.float32).at[:W1.shape[0]].set(W1)
    xp = jnp.zeros((N, 8), jnp.float32).at[:, :x.shape[1]].set(x)
    We1p = jnp.zeros((8, EMB), jnp.float32).at[:We1.shape[0]].set(We1)
    eap = jnp.zeros((E, 8), jnp.float32).at[:, :edge_attr.shape[1]].set(edge_attr)
    W5p = jnp.zeros((W5.shape[0], 8), jnp.float32).at[:, :1].set(W5)
    b5p = jnp.zeros((1, 8), jnp.float32).at[0, 0].set(b5[0])

    def r1(v):
        return v.reshape(1, -1)

    h2 = _row_call(_encode_nodes_body, N, BN, [(xp, 8, 0)],
                   [W1p, r1(b1), W2, r1(b2)], [2 * EMB])
    ea, eac = _row_call(_encode_edges_body, E, BE, [(eap, 8, 0)],
                        [We1p, r1(be1), We2, r1(be2), C2, r1(bem1)],
                        [EMB, EMB])

    for _ in range(3):
        ps, pd = _row_call(_proj_body, N, BN, [(h2, 2 * EMB, 0)],
                           [Wsd, B], [2 * EMB, EMB])
        gs = ps[send]
        gd = pd[recv]
        ea, m = _row_call(
            _edge_mlp_body, E, BE,
            [(gs, 2 * EMB, 0), (gd, EMB, 0), (ea, EMB, 0), (eac, EMB, 0)],
            [C1, Wem2, r1(bem2), F, r1(bnm1a), Wnm1b, r1(bnm1b)],
            [EMB, EMB])
        agg = jax.ops.segment_sum(m, recv, num_segments=N)
        h2 = _row_call(_node_update_body, N, BN,
                       [(h2, 2 * EMB, 0), (agg, EMB, 0)],
                       [G, H, r1(bnm2a), Wnm2b, r1(bnm2b)], [2 * EMB])

    out8 = _row_call(_decoder_body, N, BN, [(h2, 2 * EMB, 0)],
                     [W3, r1(b3), W4, r1(b4), W5p, b5p], [8])
    return out8[:, :1]


# speedup vs baseline: 1.4818x; 1.4818x over previous
import functools

import jax
import jax.numpy as jnp
from jax import lax
from jax.experimental import pallas as pl
from jax.experimental.pallas import tpu as pltpu
from jax.experimental.pallas import tpu_sc as plsc

EMB = 64
_NC = 2
_NS = 16
_CH = 128


def _relu(v):
    return jnp.maximum(v, 0.0)


def _dot(a, b):
    return lax.dot_general(a, b, (((1,), (0,)), ((), ())),
                           preferred_element_type=jnp.float32)


def _encode_nodes_body(x_ref, w1_ref, b1_ref, w2_ref, b2_ref, h2_ref):
    h = _relu(_dot(x_ref[...], w1_ref[...]) + b1_ref[...])
    h = _relu(_dot(h, w2_ref[...]) + b2_ref[...])
    h2_ref[:, :EMB] = h
    h2_ref[:, EMB:] = h


def _encode_edges_body(a_ref, w1_ref, b1_ref, w2_ref, b2_ref, c2_ref, bc_ref,
                       ea_ref):
    e = _relu(_dot(a_ref[...], w1_ref[...]) + b1_ref[...])
    e = _relu(_dot(e, w2_ref[...]) + b2_ref[...])
    ea_ref[:, :EMB] = e
    ea_ref[:, EMB:] = _dot(e, c2_ref[...]) + bc_ref[...]


def _proj_body(h2_ref, wsd_ref, wb_ref, ps_ref, pd_ref):
    h2 = h2_ref[...]
    ps_ref[...] = _dot(h2, wsd_ref[...])
    pd_ref[:, :EMB] = _dot(h2, wb_ref[...])
    pd_ref[:, EMB:] = jnp.zeros_like(h2[:, EMB:])


def _edge_mlp_body(gs_ref, gd_ref, ea_ref,
                   c1_ref, w2_ref, b2_ref, f_ref, b3_ref, w4_ref, b4_ref,
                   eao_ref, m_ref):
    gs = gs_ref[...]
    ea_in = ea_ref[...]
    t = _relu(gs[:, :EMB] + gd_ref[:, :EMB]
              + _dot(ea_in[:, :EMB], c1_ref[...]) + ea_in[:, EMB:])
    ea_new = _relu(_dot(t, w2_ref[...]) + b2_ref[...])
    u = _relu(gs[:, EMB:] + _dot(ea_new, f_ref[...]) + b3_ref[...])
    m_ref[...] = _relu(_dot(u, w4_ref[...]) + b4_ref[...])
    eao_ref[:, :EMB] = ea_new
    eao_ref[:, EMB:] = ea_in[:, EMB:]


def _node_update_body(h2_ref, agg_ref, wg_ref, wh_ref, ba_ref, w2_ref, b2_ref,
                      h2o_ref):
    z = _relu(_dot(h2_ref[...], wg_ref[...])
              + _dot(agg_ref[...], wh_ref[...]) + ba_ref[...])
    hn = _relu(_dot(z, w2_ref[...]) + b2_ref[...])
    h2o_ref[:, :EMB] = hn
    h2o_ref[:, EMB:] = h2_ref[:, EMB:]


def _decoder_body(h_ref, w3_ref, b3_ref, w4_ref, b4_ref, w5_ref, b5_ref,
                  o_ref):
    o = _relu(_dot(h_ref[:, :EMB], w3_ref[...]) + b3_ref[...])
    o = _relu(_dot(o, w4_ref[...]) + b4_ref[...])
    o_ref[...] = _dot(o, w5_ref[...]) + b5_ref[...]


def _sc_gather(ps, pd, send, recv):
    E = send.shape[0]
    nch = E // _CH
    assert E % _CH == 0
    nw = _NC * _NS
    full, extra = divmod(nch, nw)
    mesh = plsc.VectorSubcoreMesh(core_axis_name="c", subcore_axis_name="s")

    @functools.partial(
        pl.kernel, mesh=mesh,
        out_type=[jax.ShapeDtypeStruct((E, 2 * EMB), jnp.float32),
                  jax.ShapeDtypeStruct((E, 2 * EMB), jnp.float32)],
        scratch_types=[pltpu.VMEM((1, _CH), jnp.int32),
                       pltpu.VMEM((1, _CH), jnp.int32),
                       pltpu.VMEM((_CH, 2 * EMB), jnp.float32),
                       pltpu.VMEM((_CH, 2 * EMB), jnp.float32),
                       pltpu.SemaphoreType.DMA],
    )
    def k(ps_hbm, pd_hbm, send_hbm, recv_hbm, gs_hbm, gd_hbm,
          idxs_v, idxr_v, bufs_v, bufd_v, sem):
        c = lax.axis_index("c")
        s = lax.axis_index("s")
        w = s * _NC + c
        nk = jnp.where(w < extra, full + 1, full)

        def body(i, carry):
            off = (w + i * nw) * _CH
            pltpu.sync_copy(send_hbm.at[pl.ds(off, _CH)], idxs_v.at[0])
            pltpu.sync_copy(recv_hbm.at[pl.ds(off, _CH)], idxr_v.at[0])
            pltpu.async_copy(ps_hbm.at[idxs_v.at[0]], bufs_v, sem).wait()
            pltpu.async_copy(pd_hbm.at[idxr_v.at[0]], bufd_v, sem).wait()
            pltpu.sync_copy(bufs_v, gs_hbm.at[pl.ds(off, _CH)])
            pltpu.sync_copy(bufd_v, gd_hbm.at[pl.ds(off, _CH)])
            return carry

        lax.fori_loop(0, nk, body, 0)

    return k(ps, pd, send, recv)


_RNG_N = 8448
_NPASS = 3


def _sc_scatter_add(m, loc, zrows, n_nodes):
    E = m.shape[0]
    nch = E // _CH
    assert E % _CH == 0
    rng_n = _RNG_N
    acc_rows = 8576
    zpt = acc_rows // _NS
    wpt = rng_n // _NS
    nrng = _NC * _NPASS
    assert nrng * rng_n >= n_nodes
    full, extra = divmod(nch, _NS)
    mesh = plsc.VectorSubcoreMesh(core_axis_name="c", subcore_axis_name="s")

    @functools.partial(
        pl.kernel, mesh=mesh,
        out_type=jax.ShapeDtypeStruct((nrng * rng_n, EMB), jnp.float32),
        scratch_types=[pltpu.VMEM((_CH,), jnp.int32),
                       pltpu.VMEM((_CH, EMB), jnp.float32),
                       pltpu.VMEM_SHARED((acc_rows, EMB), jnp.float32)],
    )
    def k(m_hbm, loc_hbm, z_hbm, agg_hbm, idx_v, mbuf_v, acc):
        c = lax.axis_index("c")
        s = lax.axis_index("s")
        nk = jnp.where(s < extra, full + 1, full)
        for p in range(_NPASS):
            r = c * _NPASS + p
            base = r * rng_n
            pltpu.sync_copy(z_hbm.at[pl.ds(s * zpt, zpt)],
                            acc.at[pl.ds(s * zpt, zpt)])
            plsc.subcore_barrier()

            def body(i, carry):
                off = (s + i * _NS) * _CH
                pltpu.sync_copy(loc_hbm.at[pl.ds(r * E + off, _CH)], idx_v)
                pltpu.sync_copy(m_hbm.at[pl.ds(off, _CH)], mbuf_v)
                pltpu.sync_copy(mbuf_v, acc.at[idx_v], add=True)
                return carry

            lax.fori_loop(0, nk, body, 0)
            plsc.subcore_barrier()
            pltpu.sync_copy(acc.at[pl.ds(s * wpt, wpt)],
                            agg_hbm.at[pl.ds(base + s * wpt, wpt)])
            plsc.subcore_barrier()

    return k(m, loc, zrows)[:n_nodes]


def _row_call(body, n, bn, blocked, weights, out_cols):
    grid = (n // bn,)
    in_specs = [pl.BlockSpec((bn, bc), (lambda ci: (lambda i: (i, ci)))(ci))
                for (_, bc, ci) in blocked]
    in_specs += [pl.BlockSpec(w.shape, (lambda nd: (lambda i: (0,) * nd))(w.ndim))
                 for w in weights]
    out_specs = [pl.BlockSpec((bn, c), lambda i: (i, 0)) for c in out_cols]
    out_shape = [jax.ShapeDtypeStruct((n, c), jnp.float32) for c in out_cols]
    single = len(out_cols) == 1
    res = pl.pallas_call(
        body,
        grid=grid,
        in_specs=in_specs,
        out_specs=out_specs[0] if single else out_specs,
        out_shape=out_shape[0] if single else out_shape,
    )(*[a for (a, _, _) in blocked], *weights)
    return res


def kernel(x, edge_index, edge_attr, params):
    N = x.shape[0]
    E = edge_index.shape[1]
    BN = 2000 if N % 2000 == 0 else N
    BE = 2000 if E % 2000 == 0 else E
    send = edge_index[0]
    recv = edge_index[1]

    W1, b1 = params['fc1']
    W2, b2 = params['fc2']
    We1, be1 = params['efc1']
    We2, be2 = params['efc2']
    Wem1, bem1 = params['em1']
    Wem2, bem2 = params['em2']
    Wnm1a, bnm1a = params['nm1a']
    Wnm1b, bnm1b = params['nm1b']
    Wnm2a, bnm2a = params['nm2a']
    Wnm2b, bnm2b = params['nm2b']
    W3, b3 = params['fc3']
    W4, b4 = params['fc4']
    W5, b5 = params['fc5']

    A, B, C1, C2 = Wem1[0:128], Wem1[128:256], Wem1[256:320], Wem1[320:384]
    D, F = Wnm1a[0:128], Wnm1a[128:192]
    Wsd = jnp.concatenate([A, D], axis=1)
    G, H = Wnm2a[0:128], Wnm2a[128:192]

    W1p = jnp.zeros((8, EMB), jnp.float32).at[:W1.shape[0]].set(W1)
    xp = jnp.zeros((N, 8), jnp.float32).at[:, :x.shape[1]].set(x)
    We1p = jnp.zeros((8, EMB), jnp.float32).at[:We1.shape[0]].set(We1)
    eap = jnp.zeros((E, 8), jnp.float32).at[:, :edge_attr.shape[1]].set(edge_attr)
    W5p = jnp.zeros((W5.shape[0], 8), jnp.float32).at[:, :1].set(W5)
    b5p = jnp.zeros((1, 8), jnp.float32).at[0, 0].set(b5[0])

    def r1(v):
        return v.reshape(1, -1)

    h2 = _row_call(_encode_nodes_body, N, BN, [(xp, 8, 0)],
                   [W1p, r1(b1), W2, r1(b2)], [2 * EMB])
    ea = _row_call(_encode_edges_body, E, BE, [(eap, 8, 0)],
                   [We1p, r1(be1), We2, r1(be2), C2, r1(bem1)],
                   [2 * EMB])

    zrows = jnp.zeros((8576, EMB), jnp.float32)
    r_ids = jnp.arange(_NC * _NPASS, dtype=jnp.int32)[:, None]
    loc = jnp.where(recv[None, :] // _RNG_N == r_ids,
                    recv[None, :] - r_ids * _RNG_N, _RNG_N)
    loc = loc.astype(jnp.int32).reshape(-1)

    def layer(carry, _):
        h2, ea = carry
        ps, pd = _row_call(_proj_body, N, BN, [(h2, 2 * EMB, 0)],
                           [Wsd, B], [2 * EMB, 2 * EMB])
        gs, gd = _sc_gather(ps, pd, send, recv)
        ea2, m = _row_call(
            _edge_mlp_body, E, BE,
            [(gs, 2 * EMB, 0), (gd, 2 * EMB, 0), (ea, 2 * EMB, 0)],
            [C1, Wem2, r1(bem2), F, r1(bnm1a), Wnm1b, r1(bnm1b)],
            [2 * EMB, EMB])
        agg = jax.ops.segment_sum(m, recv, num_segments=N)
        h2n = _row_call(_node_update_body, N, BN,
                        [(h2, 2 * EMB, 0), (agg, EMB, 0)],
                        [G, H, r1(bnm2a), Wnm2b, r1(bnm2b)], [2 * EMB])
        return (h2n, ea2), None

    (h2, ea), _ = lax.scan(layer, (h2, ea), None, length=3)

    out8 = _row_call(_decoder_body, N, BN, [(h2, 2 * EMB, 0)],
                     [W3, r1(b3), W4, r1(b4), W5p, b5p], [8])
    return out8[:, :1]
